# baseline (speedup 1.0000x reference)
import functools
import numpy as np
import jax
import jax.numpy as jnp
from jax import lax
from jax.experimental import pallas as pl
from jax.experimental.pallas import tpu as pltpu
from jax.experimental.pallas import tpu_sc as plsc

_R = 128
_N = 16384
_B = 4
_HALF = (_R // 2) * _R * _R
_PPT = _N // 16
_STEPS = _PPT // 16
_SLICE = _HALF // 16
_ENT = _PPT * 4
_ENT2 = 2 * _ENT
_RB = 8192
_NCH = _SLICE // _RB
_CLIP_HI = float(np.float32(_R - 1 - 1e-4))


def _sc_body(pred, gt, out, cbig, ia0, va0, ib0, vb0,
             zbuf, r0, r1, accb, grid, semc, semz, sems, semr):
    c = lax.axis_index("c")
    s = lax.axis_index("s")

    hc = []
    for cl, cld in enumerate((pred, gt)):
        for b in range(_B):
            for d in range(3):
                hc.append(pltpu.async_copy(
                    cld.at[pl.ds(d * (_B * _N) + b * _N + s * _PPT, _PPT)],
                    cbig.at[pl.ds(((cl * _B + b) * 3 + d) * _PPT, _PPT)],
                    semc))

    def _mkzero(i, _):
        zbuf[pl.ds(i * 16, 16)] = jnp.zeros((16,), jnp.float32)
        return 0
    lax.fori_loop(0, _RB // 16, _mkzero, 0)

    hz = []
    for j in range(_NCH):
        hz.append(pltpu.async_copy(
            zbuf, grid.at[pl.ds(s * _SLICE + j * _RB, _RB)], semz))

    for h in hc:
        h.wait()

    def _compute(b, stg):
        with jax.named_scope("compute"):
            for cl in (0, 1):
                ist, wst = stg[2 * cl], stg[2 * cl + 1]
                sign = 1.0 if cl == 0 else -1.0
                cb = (cl * _B + b) * 3 * _PPT
                eb = 0

                def _step(i, _):
                    o = i * 16
                    xx = cbig[pl.ds(cb + o, 16)]
                    yy = cbig[pl.ds(cb + _PPT + o, 16)]
                    zz = cbig[pl.ds(cb + 2 * _PPT + o, 16)]
                    px = jnp.clip((xx + 1.0) * 0.5 * (_R - 1), 0.0, _CLIP_HI)
                    py = jnp.clip((yy + 1.0) * 0.5 * (_R - 1), 0.0, _CLIP_HI)
                    pz = jnp.clip((zz + 1.0) * 0.5 * (_R - 1), 0.0, _CLIP_HI)
                    ix = px.astype(jnp.int32)
                    iy = py.astype(jnp.int32)
                    iz = pz.astype(jnp.int32)
                    fx = px - ix.astype(jnp.float32)
                    fy = py - iy.astype(jnp.float32)
                    fz = pz - iz.astype(jnp.float32)
                    pmask = (ix & 1) == c
                    wx1 = fx * sign
                    wx0 = sign - wx1
                    wxc = jnp.where(pmask, wx0, wx1)
                    gxc = jnp.where(pmask, ix, ix + 1)
                    base = (gxc >> 1) * (_R * _R) + iy * _R + iz
                    wy0 = 1.0 - fy
                    wz0 = 1.0 - fz
                    wxy0 = wxc * wy0
                    wxy1 = wxc * fy
                    colb = eb + i * 64
                    for k, (dy, dz) in enumerate(
                            ((0, 0), (0, 1), (1, 0), (1, 1))):
                        wst[pl.ds(colb + k * 16, 16)] = (
                            (wxy1 if dy else wxy0) * (fz if dz else wz0))
                        ist[pl.ds(colb + k * 16, 16)] = base + (dy * _R + dz)
                    return 0
                lax.fori_loop(0, _STEPS, _step, 0)

    stage = (ia0, va0, ib0, vb0)
    acc = jnp.zeros((16,), jnp.float32)

    for b in range(_B):
        ia, va, ib, vb = stage
        _compute(b, stage)

        for h in hz:
            h.wait()
        plsc.subcore_barrier()

        with jax.named_scope("scatter"):
            h0 = pltpu.async_copy(va, grid.at[ia], sems, add=True)
            h1 = pltpu.async_copy(vb, grid.at[ib], sems, add=True)
            h0.wait()
            h1.wait()
        plsc.subcore_barrier()

        with jax.named_scope("reduce"):
            hz = []
            sbase = s * _SLICE
            for j in range(_NCH):
                pltpu.sync_copy(grid.at[pl.ds(sbase + j * _RB, _RB)], r0)

                def _inner(t, aa):
                    tb = t * 128
                    p = []
                    for u in range(8):
                        p.append(jnp.abs(r0[pl.ds(tb + u * 16, 16)]))
                    return aa + (((p[0] + p[1]) + (p[2] + p[3]))
                                 + ((p[4] + p[5]) + (p[6] + p[7])))
                acc = lax.fori_loop(0, _RB // 128, _inner, acc)
            if b + 1 < _B:
                for j in range(_NCH):
                    hz.append(pltpu.async_copy(
                        zbuf, grid.at[pl.ds(sbase + j * _RB, _RB)], semz))

    accb[...] = acc
    pltpu.sync_copy(accb, out.at[c, s])


@functools.partial(
    pl.kernel,
    out_type=jax.ShapeDtypeStruct((2, 16, 16), jnp.float32),
    mesh=plsc.VectorSubcoreMesh(core_axis_name="c", subcore_axis_name="s"),
    scratch_types=[
        pltpu.VMEM((24 * _PPT,), jnp.float32),
        pltpu.VMEM((_ENT,), jnp.int32),
        pltpu.VMEM((_ENT,), jnp.float32),
        pltpu.VMEM((_ENT,), jnp.int32),
        pltpu.VMEM((_ENT,), jnp.float32),
        pltpu.VMEM((_RB,), jnp.float32),
        pltpu.VMEM((_RB,), jnp.float32),
        pltpu.VMEM((_RB,), jnp.float32),
        pltpu.VMEM((16,), jnp.float32),
        pltpu.VMEM_SHARED((_HALF,), jnp.float32),
        pltpu.SemaphoreType.DMA,
        pltpu.SemaphoreType.DMA,
        pltpu.SemaphoreType.DMA,
        pltpu.SemaphoreType.DMA,
    ],
)
def _gridding_dist_sc(pred, gt, out, *scratch):
    _sc_body(pred, gt, out, *scratch)


@jax.jit
def kernel(pred_cloud, gt_cloud):
    partials = _gridding_dist_sc(
        pred_cloud.transpose(2, 0, 1).reshape(-1),
        gt_cloud.transpose(2, 0, 1).reshape(-1))
    return jnp.sum(partials) / (_B * _R ** 3)


# speedup vs baseline: 1.0113x; 1.0113x over previous
import functools
import numpy as np
import jax
import jax.numpy as jnp
from jax import lax
from jax.experimental import pallas as pl
from jax.experimental.pallas import tpu as pltpu
from jax.experimental.pallas import tpu_sc as plsc

_R = 128
_N = 16384
_B = 4
_HALF = (_R // 2) * _R * _R
_PPT = _N // 16
_STEPS = _PPT // 16
_SLICE = _HALF // 16
_ENT = _PPT * 4
_ENT2 = 2 * _ENT
_RB = 8192
_NCH = _SLICE // _RB
_CLIP_HI = float(np.float32(_R - 1 - 1e-4))


def _sc_body(pred, gt, out, cbig, iaA, vaA, ibA, vbA, iaB, vaB, ibB, vbB,
             zbuf, r0, accb, grid, semc, semz, sems):
    c = lax.axis_index("c")
    s = lax.axis_index("s")

    hc = []
    for cl, cld in enumerate((pred, gt)):
        for b in range(_B):
            for d in range(3):
                hc.append(pltpu.async_copy(
                    cld.at[pl.ds(d * (_B * _N) + b * _N + s * _PPT, _PPT)],
                    cbig.at[pl.ds(((cl * _B + b) * 3 + d) * _PPT, _PPT)],
                    semc))

    def _mkzero(i, _):
        zbuf[pl.ds(i * 16, 16)] = jnp.zeros((16,), jnp.float32)
        return 0
    lax.fori_loop(0, _RB // 16, _mkzero, 0)

    hz = []
    for j in range(_NCH):
        hz.append(pltpu.async_copy(
            zbuf, grid.at[pl.ds(s * _SLICE + j * _RB, _RB)], semz))

    for h in hc:
        h.wait()

    def _compute(b, half, stg):
        with jax.named_scope("compute"):
            for cl in (0, 1):
                ist, wst = stg[2 * cl], stg[2 * cl + 1]
                sign = 1.0 if cl == 0 else -1.0
                cb = (cl * _B + b) * 3 * _PPT
                eb = -half * (_ENT // 2)

                def _step(i, _):
                    o = i * 16
                    xx = cbig[pl.ds(cb + o, 16)]
                    yy = cbig[pl.ds(cb + _PPT + o, 16)]
                    zz = cbig[pl.ds(cb + 2 * _PPT + o, 16)]
                    px = jnp.clip((xx + 1.0) * 0.5 * (_R - 1), 0.0, _CLIP_HI)
                    py = jnp.clip((yy + 1.0) * 0.5 * (_R - 1), 0.0, _CLIP_HI)
                    pz = jnp.clip((zz + 1.0) * 0.5 * (_R - 1), 0.0, _CLIP_HI)
                    ix = px.astype(jnp.int32)
                    iy = py.astype(jnp.int32)
                    iz = pz.astype(jnp.int32)
                    fx = px - ix.astype(jnp.float32)
                    fy = py - iy.astype(jnp.float32)
                    fz = pz - iz.astype(jnp.float32)
                    pmask = (ix & 1) == c
                    wx1 = fx * sign
                    wx0 = sign - wx1
                    wxc = jnp.where(pmask, wx0, wx1)
                    gxc = jnp.where(pmask, ix, ix + 1)
                    base = (gxc >> 1) * (_R * _R) + iy * _R + iz
                    wy0 = 1.0 - fy
                    wz0 = 1.0 - fz
                    wxy0 = wxc * wy0
                    wxy1 = wxc * fy
                    colb = eb + i * 64
                    for k, (dy, dz) in enumerate(
                            ((0, 0), (0, 1), (1, 0), (1, 1))):
                        wst[pl.ds(colb + k * 16, 16)] = (
                            (wxy1 if dy else wxy0) * (fz if dz else wz0))
                        ist[pl.ds(colb + k * 16, 16)] = base + (dy * _R + dz)
                    return 0
                lax.fori_loop(half * (_STEPS // 2),
                              (half + 1) * (_STEPS // 2), _step, 0)

    stageA = (iaA, vaA, ibA, vbA)
    stageB = (iaB, vaB, ibB, vbB)
    acc = jnp.zeros((16,), jnp.float32)

    for b in range(_B):
        _compute(b, 0, stageA)

        for h in hz:
            h.wait()
        plsc.subcore_barrier()

        with jax.named_scope("scatter"):
            h0 = pltpu.async_copy(vaA, grid.at[iaA], sems, add=True)
            h1 = pltpu.async_copy(vbA, grid.at[ibA], sems, add=True)
            _compute(b, 1, stageB)
            h2 = pltpu.async_copy(vaB, grid.at[iaB], sems, add=True)
            h3 = pltpu.async_copy(vbB, grid.at[ibB], sems, add=True)
            h0.wait()
            h1.wait()
            h2.wait()
            h3.wait()
        plsc.subcore_barrier()

        with jax.named_scope("reduce"):
            hz = []
            sbase = s * _SLICE
            for j in range(_NCH):
                pltpu.sync_copy(grid.at[pl.ds(sbase + j * _RB, _RB)], r0)

                def _inner(t, aa):
                    tb = t * 128
                    p = []
                    for u in range(8):
                        p.append(jnp.abs(r0[pl.ds(tb + u * 16, 16)]))
                    return aa + (((p[0] + p[1]) + (p[2] + p[3]))
                                 + ((p[4] + p[5]) + (p[6] + p[7])))
                acc = lax.fori_loop(0, _RB // 128, _inner, acc)
            if b + 1 < _B:
                for j in range(_NCH):
                    hz.append(pltpu.async_copy(
                        zbuf, grid.at[pl.ds(sbase + j * _RB, _RB)], semz))

    accb[...] = acc
    pltpu.sync_copy(accb, out.at[c, s])


@functools.partial(
    pl.kernel,
    out_type=jax.ShapeDtypeStruct((2, 16, 16), jnp.float32),
    mesh=plsc.VectorSubcoreMesh(core_axis_name="c", subcore_axis_name="s"),
    scratch_types=[
        pltpu.VMEM((24 * _PPT,), jnp.float32),
        pltpu.VMEM((_ENT // 2,), jnp.int32),
        pltpu.VMEM((_ENT // 2,), jnp.float32),
        pltpu.VMEM((_ENT // 2,), jnp.int32),
        pltpu.VMEM((_ENT // 2,), jnp.float32),
        pltpu.VMEM((_ENT // 2,), jnp.int32),
        pltpu.VMEM((_ENT // 2,), jnp.float32),
        pltpu.VMEM((_ENT // 2,), jnp.int32),
        pltpu.VMEM((_ENT // 2,), jnp.float32),
        pltpu.VMEM((_RB,), jnp.float32),
        pltpu.VMEM((_RB,), jnp.float32),
        pltpu.VMEM((16,), jnp.float32),
        pltpu.VMEM_SHARED((_HALF,), jnp.float32),
        pltpu.SemaphoreType.DMA,
        pltpu.SemaphoreType.DMA,
        pltpu.SemaphoreType.DMA,
    ],
)
def _gridding_dist_sc(pred, gt, out, *scratch):
    _sc_body(pred, gt, out, *scratch)


@jax.jit
def kernel(pred_cloud, gt_cloud):
    partials = _gridding_dist_sc(
        pred_cloud.transpose(2, 0, 1).reshape(-1),
        gt_cloud.transpose(2, 0, 1).reshape(-1))
    return jnp.sum(partials) / (_B * _R ** 3)
